# baseline (speedup 1.0000x reference)
import functools

import jax
import jax.numpy as jnp
from jax import lax
from jax.experimental import pallas as pl
from jax.experimental.pallas import tpu as pltpu
from jax.experimental.pallas import tpu_sc as plsc

EMB = 64
TPAD = 128
FPAD = 256
NC, NS = 2, 16
NW = NC * NS
LANES = 16
NREP = 16
BTILE = 128
NPH = 4


def _proj_body(t_ref, w_ref, b_ref, out_ref):
    out_ref[...] = (
        jnp.dot(t_ref[...], w_ref[...], preferred_element_type=jnp.float32)
        + b_ref[...]
    )


def _packed_table(table, W, b):
    tp = jnp.zeros((TPAD, FPAD), jnp.float32).at[: table.shape[0], : table.shape[1]].set(table)
    wp = jnp.zeros((FPAD, EMB), jnp.float32).at[: W.shape[0]].set(W)
    proj = pl.pallas_call(
        _proj_body,
        out_shape=jax.ShapeDtypeStruct((TPAD, EMB), jnp.float32),
    )(tp, wp, b.reshape(1, EMB))
    u = jax.lax.bitcast_convert_type(proj.astype(jnp.bfloat16), jnp.uint16)
    w32 = (u[:, 1::2].astype(jnp.uint32) << 16) | u[:, 0::2].astype(jnp.uint32)
    w32 = w32.T
    tab8 = jnp.broadcast_to(w32[:, :, None], (EMB // 2, TPAD, NREP))
    return jax.lax.bitcast_convert_type(
        tab8.reshape(EMB // 2 * TPAD * NREP), jnp.int32
    )


@functools.lru_cache(maxsize=None)
def _make_lookup(B, L):
    assert B % (NW * NPH * BTILE) == 0
    b_per_w = B // NW
    b_per_ph = b_per_w // NPH
    n_bt = b_per_ph // BTILE
    n_lt = L * n_bt
    assert n_lt % 2 == 0
    mesh = plsc.VectorSubcoreMesh(core_axis_name="c", subcore_axis_name="s")

    @functools.partial(
        pl.kernel,
        out_type=jax.ShapeDtypeStruct((L, EMB, B), jnp.float32),
        mesh=mesh,
        scratch_types=[
            pltpu.VMEM((b_per_ph, L), jnp.int32),
            pltpu.VMEM((EMB // 2 * TPAD * NREP,), jnp.int32),
            pltpu.VMEM((2, EMB, BTILE), jnp.float32),
            pltpu.SemaphoreType.DMA,
            pltpu.SemaphoreType.DMA,
        ],
        compiler_params=pltpu.CompilerParams(needs_layout_passes=False),
    )
    def lookup(tab_hbm, idx_hbm, out_hbm, idx_v, tab_v, tiles_v, s0, s1):
        wid = lax.axis_index("s") * NC + lax.axis_index("c")
        b0 = wid * b_per_w
        sems = (s0, s1)
        pltpu.sync_copy(tab_hbm, tab_v)
        lane = lax.iota(jnp.int32, LANES)
        lrep = lane & (NREP - 1)

        def fill_slab(h, s, p):
            l = s // n_bt
            t = s % n_bt

            @pl.loop(0, BTILE // LANES)
            def _(k):
                rows = t * BTILE + k * LANES + lane
                vidx = plsc.load_gather(idx_v, [rows, jnp.broadcast_to(l, (LANES,))])
                vbase = (vidx << 4) | lrep
                kcol = pl.ds(k * LANES, LANES)
                pend = []
                for ep in range(EMB // 2):
                    seg = tab_v.at[pl.ds(ep * TPAD * NREP, TPAD * NREP)]
                    x = plsc.load_gather(seg, [vbase])
                    lo = plsc.bitcast(x << 16, jnp.float32)
                    hi = plsc.bitcast(x, jnp.float32)
                    pend.append((ep, lo, hi))
                    if len(pend) > 2:
                        pep, plo, phi = pend.pop(0)
                        tiles_v[p, 2 * pep, kcol] = plo
                        tiles_v[p, 2 * pep + 1, kcol] = phi
                for pep, plo, phi in pend:
                    tiles_v[p, 2 * pep, kcol] = plo
                    tiles_v[p, 2 * pep + 1, kcol] = phi

        def slab_out_desc(h, s, p):
            l = s // n_bt
            t = s % n_bt
            return pltpu.make_async_copy(
                tiles_v.at[p],
                out_hbm.at[l, :, pl.ds(b0 + h * b_per_ph + t * BTILE, BTILE)],
                sems[p],
            )

        for h in range(NPH):
            pltpu.sync_copy(idx_hbm.at[pl.ds(b0 + h * b_per_ph, b_per_ph)], idx_v)

            for p in range(2):
                fill_slab(h, p, p)
                slab_out_desc(h, p, p).start()

            @pl.loop(2, n_lt, step=2)
            def _(g0):
                for p in range(2):
                    s = g0 + p
                    slab_out_desc(h, s - 2, p).wait()
                    fill_slab(h, s, p)
                    slab_out_desc(h, s, p).start()

            for p in range(2):
                slab_out_desc(h, n_lt - 2 + p, p).wait()

    return lookup


def kernel(input, table, W, b):
    B, L = input.shape
    tab8 = _packed_table(table, W, b)
    out_t = _make_lookup(B, L)(tab8, input.astype(jnp.int32))
    return jnp.transpose(out_t, (2, 0, 1))


# speedup vs baseline: 1.0618x; 1.0618x over previous
import functools

import jax
import jax.numpy as jnp
from jax import lax
from jax.experimental import pallas as pl
from jax.experimental.pallas import tpu as pltpu
from jax.experimental.pallas import tpu_sc as plsc

EMB = 64
TPAD = 128
FPAD = 256
NC, NS = 2, 16
NW = NC * NS
LANES = 16
NREP = 8
BTILE = 128


def _proj_body(t_ref, w_ref, b_ref, out_ref):
    out_ref[...] = (
        jnp.dot(t_ref[...], w_ref[...], preferred_element_type=jnp.float32)
        + b_ref[...]
    )


def _packed_table(table, W, b):
    tp = jnp.zeros((TPAD, FPAD), jnp.float32).at[: table.shape[0], : table.shape[1]].set(table)
    wp = jnp.zeros((FPAD, EMB), jnp.float32).at[: W.shape[0]].set(W)
    proj = pl.pallas_call(
        _proj_body,
        out_shape=jax.ShapeDtypeStruct((TPAD, EMB), jnp.float32),
    )(tp, wp, b.reshape(1, EMB))
    u = jax.lax.bitcast_convert_type(proj.astype(jnp.bfloat16), jnp.uint16)
    w32 = (u[:, 1::2].astype(jnp.uint32) << 16) | u[:, 0::2].astype(jnp.uint32)
    w32 = w32.T
    tab8 = jnp.broadcast_to(w32[:, :, None], (EMB // 2, TPAD, NREP))
    return jax.lax.bitcast_convert_type(
        tab8.reshape(EMB // 2 * TPAD * NREP), jnp.int32
    )


@functools.lru_cache(maxsize=None)
def _make_lookup(B, L):
    assert B % (NW * BTILE) == 0
    b_per_w = B // NW
    n_bt = b_per_w // BTILE
    n_lt = L * n_bt
    assert n_lt % 2 == 0
    mesh = plsc.VectorSubcoreMesh(core_axis_name="c", subcore_axis_name="s")

    @functools.partial(
        pl.kernel,
        out_type=jax.ShapeDtypeStruct((L, EMB, B), jnp.float32),
        mesh=mesh,
        scratch_types=[
            pltpu.VMEM((b_per_w, L), jnp.int32),
            pltpu.VMEM((EMB // 2 * TPAD * NREP,), jnp.int32),
            pltpu.VMEM((2, EMB, BTILE), jnp.float32),
            pltpu.SemaphoreType.DMA,
            pltpu.SemaphoreType.DMA,
        ],
        compiler_params=pltpu.CompilerParams(needs_layout_passes=False),
    )
    def lookup(tab_hbm, idx_hbm, out_hbm, idx_v, tab_v, tiles_v, s0, s1):
        wid = lax.axis_index("s") * NC + lax.axis_index("c")
        b0 = wid * b_per_w
        sems = (s0, s1)
        pltpu.sync_copy(tab_hbm, tab_v)
        pltpu.sync_copy(idx_hbm.at[pl.ds(b0, b_per_w)], idx_v)
        lane = lax.iota(jnp.int32, LANES)
        lrep = lane & (NREP - 1)
        himask = jnp.broadcast_to(jnp.int32(-65536), (LANES,))

        def fill_slab(s, p):
            l = s // n_bt
            t = s % n_bt

            @pl.loop(0, BTILE // LANES)
            def _(k):
                rows = t * BTILE + k * LANES + lane
                vidx = plsc.load_gather(idx_v, [rows, jnp.broadcast_to(l, (LANES,))])
                vbase = (vidx << 3) | lrep
                kcol = pl.ds(k * LANES, LANES)
                pend = []
                for ep in range(EMB // 2):
                    seg = tab_v.at[pl.ds(ep * TPAD * NREP, TPAD * NREP)]
                    x = plsc.load_gather(seg, [vbase])
                    lo = plsc.bitcast(x << 16, jnp.float32)
                    hi = plsc.bitcast(x, jnp.float32)
                    pend.append((ep, lo, hi))
                    if len(pend) > 2:
                        pep, plo, phi = pend.pop(0)
                        tiles_v[p, 2 * pep, kcol] = plo
                        tiles_v[p, 2 * pep + 1, kcol] = phi
                for pep, plo, phi in pend:
                    tiles_v[p, 2 * pep, kcol] = plo
                    tiles_v[p, 2 * pep + 1, kcol] = phi

        def slab_out_desc(s, p):
            l = s // n_bt
            t = s % n_bt
            return pltpu.make_async_copy(
                tiles_v.at[p],
                out_hbm.at[l, :, pl.ds(b0 + t * BTILE, BTILE)],
                sems[p],
            )

        for p in range(2):
            fill_slab(p, p)
            slab_out_desc(p, p).start()

        @pl.loop(2, n_lt, step=2)
        def _(g0):
            for p in range(2):
                s = g0 + p
                slab_out_desc(s - 2, p).wait()
                fill_slab(s, p)
                slab_out_desc(s, p).start()

        for p in range(2):
            slab_out_desc(n_lt - 2 + p, p).wait()

    return lookup


def kernel(input, table, W, b):
    B, L = input.shape
    tab8 = _packed_table(table, W, b)
    out_t = _make_lookup(B, L)(tab8, input.astype(jnp.int32))
    return jnp.transpose(out_t, (2, 0, 1))


# speedup vs baseline: 1.1182x; 1.0532x over previous
import functools

import jax
import jax.numpy as jnp
from jax import lax
from jax.experimental import pallas as pl
from jax.experimental.pallas import tpu as pltpu
from jax.experimental.pallas import tpu_sc as plsc

EMB = 64
TPAD = 128
FPAD = 256
NC, NS = 2, 16
NW = NC * NS
LANES = 16
NREP = 8
BTILE = 128


def _proj_body(t_ref, w_ref, b_ref, out_ref):
    out_ref[...] = (
        jnp.dot(t_ref[...], w_ref[...], preferred_element_type=jnp.float32)
        + b_ref[...]
    )


def _packed_table(table, W, b):
    tp = jnp.zeros((TPAD, FPAD), jnp.float32).at[: table.shape[0], : table.shape[1]].set(table)
    wp = jnp.zeros((FPAD, EMB), jnp.float32).at[: W.shape[0]].set(W)
    proj = pl.pallas_call(
        _proj_body,
        out_shape=jax.ShapeDtypeStruct((TPAD, EMB), jnp.float32),
    )(tp, wp, b.reshape(1, EMB))
    u = jax.lax.bitcast_convert_type(proj.astype(jnp.bfloat16), jnp.uint16)
    w32 = (u[:, 1::2].astype(jnp.uint32) << 16) | u[:, 0::2].astype(jnp.uint32)
    w32 = w32.T
    tab8 = jnp.broadcast_to(w32[:, :, None], (EMB // 2, TPAD, NREP))
    return jax.lax.bitcast_convert_type(
        tab8.reshape(EMB // 2 * TPAD * NREP), jnp.int32
    )


@functools.lru_cache(maxsize=None)
def _make_lookup(B, L):
    assert B % (NW * BTILE) == 0
    b_per_w = B // NW
    n_bt = b_per_w // BTILE
    n_lt = L * n_bt
    assert n_lt % 2 == 0
    mesh = plsc.VectorSubcoreMesh(core_axis_name="c", subcore_axis_name="s")

    @functools.partial(
        pl.kernel,
        out_type=jax.ShapeDtypeStruct((L, EMB, B), jnp.float32),
        mesh=mesh,
        scratch_types=[
            pltpu.VMEM((b_per_w, L), jnp.int32),
            pltpu.VMEM((EMB // 2 * TPAD * NREP,), jnp.int32),
            pltpu.VMEM((2, EMB, BTILE), jnp.float32),
            pltpu.SemaphoreType.DMA,
            pltpu.SemaphoreType.DMA,
        ],
        compiler_params=pltpu.CompilerParams(needs_layout_passes=False),
    )
    def lookup(tab_hbm, idx_hbm, out_hbm, idx_v, tab_v, tiles_v, s0, s1):
        wid = lax.axis_index("s") * NC + lax.axis_index("c")
        b0 = wid * b_per_w
        sems = (s0, s1)
        pltpu.sync_copy(tab_hbm, tab_v)
        pltpu.sync_copy(idx_hbm.at[pl.ds(b0, b_per_w)], idx_v)
        lane = lax.iota(jnp.int32, LANES)
        lrep = lane & (NREP - 1)
        himask = jnp.broadcast_to(jnp.int32(-65536), (LANES,))

        def fill_slab(s, p):
            l = s // n_bt
            t = s % n_bt

            @pl.loop(0, BTILE // LANES)
            def _(k):
                rows = t * BTILE + k * LANES + lane
                vidx = plsc.load_gather(idx_v, [rows, jnp.broadcast_to(l, (LANES,))])
                vbase = (vidx << 3) | lrep
                kcol = pl.ds(k * LANES, LANES)
                pend = []
                for ep in range(EMB // 2):
                    seg = tab_v.at[pl.ds(ep * TPAD * NREP, TPAD * NREP)]
                    x = plsc.load_gather(seg, [vbase])
                    lo = plsc.bitcast(x << 16, jnp.float32)
                    hi = plsc.bitcast(x, jnp.float32)
                    pend.append((ep, lo, hi))
                    if len(pend) > 4:
                        pep, plo, phi = pend.pop(0)
                        tiles_v[p, 2 * pep, kcol] = plo
                        tiles_v[p, 2 * pep + 1, kcol] = phi
                for pep, plo, phi in pend:
                    tiles_v[p, 2 * pep, kcol] = plo
                    tiles_v[p, 2 * pep + 1, kcol] = phi

        def slab_out_desc(s, p):
            l = s // n_bt
            t = s % n_bt
            return pltpu.make_async_copy(
                tiles_v.at[p],
                out_hbm.at[l, :, pl.ds(b0 + t * BTILE, BTILE)],
                sems[p],
            )

        for p in range(2):
            fill_slab(p, p)
            slab_out_desc(p, p).start()

        @pl.loop(2, n_lt, step=2)
        def _(g0):
            for p in range(2):
                s = g0 + p
                slab_out_desc(s - 2, p).wait()
                fill_slab(s, p)
                slab_out_desc(s, p).start()

        for p in range(2):
            slab_out_desc(n_lt - 2 + p, p).wait()

    return lookup


def kernel(input, table, W, b):
    B, L = input.shape
    tab8 = _packed_table(table, W, b)
    out_t = _make_lookup(B, L)(tab8, input.astype(jnp.int32))
    return jnp.transpose(out_t, (2, 0, 1))
